# Initial kernel scaffold; baseline (speedup 1.0000x reference)
import jax
import jax.numpy as jnp
from jax.experimental import pallas as pl


def kernel(x, embedding):
    raise NotImplementedError("write your pallas kernel here")


# speedup vs baseline: 1.0217x; 1.0217x over previous
import functools

import jax
import jax.numpy as jnp
from jax import lax
from jax.experimental import pallas as pl
from jax.experimental.pallas import tpu as pltpu
from jax.experimental.pallas import tpu_sc as plsc

K = 1024
D = 64
N = 18432
M = 2048
GRID = N // M

_NW = 32
_BPW = N // _NW
_CHUNK = 96
_NCH = _BPW // _CHUNK


def _vq_tc_body(x_ref, emb_ref, idx_ref, loss_ref, perp_ref, cnt_ref, acc_ref):
    i = pl.program_id(0)

    @pl.when(i == 0)
    def _init():
        cnt_ref[...] = jnp.zeros_like(cnt_ref)
        acc_ref[...] = jnp.zeros_like(acc_ref)

    x = x_ref[...]
    emb = emb_ref[...]
    s = jnp.dot(x, emb, preferred_element_type=jnp.float32)
    x2 = jnp.sum(x ** 2, axis=1, keepdims=True)
    e2 = jnp.sum(emb ** 2, axis=0, keepdims=True)
    neg = -(x2 - 2.0 * s + e2)
    m = jnp.max(neg, axis=1, keepdims=True)
    iota = lax.broadcasted_iota(jnp.int32, neg.shape, 1)
    idx = jnp.min(jnp.where(neg == m, iota, K), axis=1)
    idx_ref[...] = idx
    cnt_ref[...] += jnp.sum((iota == idx[:, None]).astype(jnp.float32),
                            axis=0, keepdims=True)
    acc_ref[...] = acc_ref[...] - jnp.sum(m, keepdims=True)

    @pl.when(i == GRID - 1)
    def _fin():
        sse = acc_ref[...]
        loss_ref[...] = (1.25 / (N * D)) * sse
        p = cnt_ref[...] * (1.0 / N)
        ent = -jnp.sum(p * jnp.log(p + 1e-10), keepdims=True)
        perp_ref[...] = jnp.exp(ent)


_tc_call = pl.pallas_call(
    _vq_tc_body,
    grid=(GRID,),
    in_specs=[
        pl.BlockSpec((M, D), lambda i: (i, 0)),
        pl.BlockSpec((D, K), lambda i: (0, 0)),
    ],
    out_specs=[
        pl.BlockSpec((M,), lambda i: (i,)),
        pl.BlockSpec((1, 1), lambda i: (0, 0)),
        pl.BlockSpec((1, 1), lambda i: (0, 0)),
    ],
    out_shape=[
        jax.ShapeDtypeStruct((N,), jnp.int32),
        jax.ShapeDtypeStruct((1, 1), jnp.float32),
        jax.ShapeDtypeStruct((1, 1), jnp.float32),
    ],
    scratch_shapes=[
        pltpu.VMEM((1, K), jnp.float32),
        pltpu.VMEM((1, 1), jnp.float32),
    ],
)


def _sc_gather(table, idx2d):
    mesh = plsc.VectorSubcoreMesh(core_axis_name="c", subcore_axis_name="s")

    @functools.partial(
        pl.kernel,
        mesh=mesh,
        compiler_params=pltpu.CompilerParams(use_tc_tiling_on_sc=False),
        out_type=jax.ShapeDtypeStruct((N, D), jnp.float32),
        scratch_types=[
            pltpu.VMEM((_NCH, _CHUNK), jnp.int32),
            pltpu.VMEM((_BPW, D), jnp.float32),
            pltpu.SemaphoreType.DMA,
        ],
    )
    def gk(table_hbm, idx_hbm, out_hbm, idx_v, rows_v, sem):
        wid = lax.axis_index("s") * 2 + lax.axis_index("c")
        pltpu.sync_copy(idx_hbm.at[wid], idx_v)
        descs = [
            pltpu.async_copy(table_hbm.at[idx_v.at[j]],
                             rows_v.at[pl.ds(j * _CHUNK, _CHUNK)], sem)
            for j in range(_NCH)
        ]
        for dsc in descs:
            dsc.wait()
        pltpu.sync_copy(rows_v, out_hbm.at[pl.ds(wid * _BPW, _BPW)])

    return gk(table, idx2d)


def kernel(x, embedding):
    flat_x = x.reshape(-1, D)
    idx, loss11, perp11 = _tc_call(flat_x, embedding)
    quantized = _sc_gather(embedding.T, idx.reshape(_NW, _NCH, _CHUNK))
    return (
        loss11[0, 0],
        quantized.reshape(x.shape),
        perp11[0, 0],
        idx.reshape(x.shape[:1] + x.shape[2:]),
    )


# speedup vs baseline: 1.0733x; 1.0504x over previous
import functools

import jax
import jax.numpy as jnp
from jax import lax
from jax.experimental import pallas as pl
from jax.experimental.pallas import tpu as pltpu
from jax.experimental.pallas import tpu_sc as plsc

K = 1024
D = 64
N = 18432
M = 2048
GRID = N // M

_NW = 32
_BPW = N // _NW
_CHUNK = 96
_NCH = _BPW // _CHUNK


def _vq_tc_body(x_ref, emb_ref, idx_ref, loss_ref, perp_ref, cnt_ref, acc_ref):
    i = pl.program_id(0)

    @pl.when(i == 0)
    def _init():
        cnt_ref[...] = jnp.zeros_like(cnt_ref)
        acc_ref[...] = jnp.zeros_like(acc_ref)

    x = x_ref[...]
    emb = emb_ref[...]
    s = jnp.dot(x, emb, preferred_element_type=jnp.float32)
    x2 = jnp.sum(x ** 2, axis=1, keepdims=True)
    e2 = jnp.sum(emb ** 2, axis=0, keepdims=True)
    d = x2 - 2.0 * s + e2
    m = jnp.min(d, axis=1, keepdims=True)
    eqm = d == m
    iota = lax.broadcasted_iota(jnp.int32, (1, K), 1).astype(jnp.float32)
    idx = jnp.min(jnp.where(eqm, iota, float(K)), axis=1)
    idx_ref[...] = idx.astype(jnp.int32)
    oh = eqm.astype(jnp.float32)
    ones = jnp.ones((1, M), jnp.float32)
    cnt_ref[...] += lax.dot_general(ones, oh, (((1,), (0,)), ((), ())),
                                    preferred_element_type=jnp.float32)
    acc_ref[...] = acc_ref[...] + jnp.sum(m, keepdims=True)

    @pl.when(i == GRID - 1)
    def _fin():
        sse = acc_ref[...]
        loss_ref[...] = (1.25 / (N * D)) * sse
        p = cnt_ref[...] * (1.0 / N)
        ent = -jnp.sum(p * jnp.log(p + 1e-10), keepdims=True)
        perp_ref[...] = jnp.exp(ent)


_tc_call = pl.pallas_call(
    _vq_tc_body,
    grid=(GRID,),
    in_specs=[
        pl.BlockSpec((M, D), lambda i: (i, 0)),
        pl.BlockSpec((D, K), lambda i: (0, 0)),
    ],
    out_specs=[
        pl.BlockSpec((M,), lambda i: (i,)),
        pl.BlockSpec((1, 1), lambda i: (0, 0)),
        pl.BlockSpec((1, 1), lambda i: (0, 0)),
    ],
    out_shape=[
        jax.ShapeDtypeStruct((N,), jnp.int32),
        jax.ShapeDtypeStruct((1, 1), jnp.float32),
        jax.ShapeDtypeStruct((1, 1), jnp.float32),
    ],
    scratch_shapes=[
        pltpu.VMEM((1, K), jnp.float32),
        pltpu.VMEM((1, 1), jnp.float32),
    ],
)


def _sc_gather(table, idx2d):
    mesh = plsc.VectorSubcoreMesh(core_axis_name="c", subcore_axis_name="s")

    @functools.partial(
        pl.kernel,
        mesh=mesh,
        compiler_params=pltpu.CompilerParams(use_tc_tiling_on_sc=False),
        out_type=jax.ShapeDtypeStruct((N, D), jnp.float32),
        scratch_types=[
            pltpu.VMEM((_NCH, _CHUNK), jnp.int32),
            pltpu.VMEM((_BPW, D), jnp.float32),
            pltpu.SemaphoreType.DMA,
        ],
    )
    def gk(table_hbm, idx_hbm, out_hbm, idx_v, rows_v, sem):
        wid = lax.axis_index("s") * 2 + lax.axis_index("c")
        pltpu.sync_copy(idx_hbm.at[wid], idx_v)
        descs = [
            pltpu.async_copy(table_hbm.at[idx_v.at[j]],
                             rows_v.at[pl.ds(j * _CHUNK, _CHUNK)], sem)
            for j in range(_NCH)
        ]
        for dsc in descs:
            dsc.wait()
        pltpu.sync_copy(rows_v, out_hbm.at[pl.ds(wid * _BPW, _BPW)])

    return gk(table, idx2d)


def kernel(x, embedding):
    flat_x = x.reshape(-1, D)
    idx, loss11, perp11 = _tc_call(flat_x, embedding)
    quantized = _sc_gather(embedding.T, idx.reshape(_NW, _NCH, _CHUNK))
    return (
        loss11[0, 0],
        quantized.reshape(x.shape),
        perp11[0, 0],
        idx.reshape(x.shape[:1] + x.shape[2:]),
    )


# speedup vs baseline: 1.2254x; 1.1417x over previous
import functools

import jax
import jax.numpy as jnp
from jax import lax
from jax.experimental import pallas as pl
from jax.experimental.pallas import tpu as pltpu
from jax.experimental.pallas import tpu_sc as plsc

K = 1024
D = 64
N = 18432
M = 2048
GRID = N // M

_NW = 32
_BPW = N // _NW
_CHUNK = 96
_NCH = _BPW // _CHUNK


def _vq_tc_body(x_ref, emb_ref, idx_ref, loss_ref, perp_ref, cnt_ref, acc_ref):
    i = pl.program_id(0)

    @pl.when(i == 0)
    def _init():
        cnt_ref[...] = jnp.zeros_like(cnt_ref)
        acc_ref[...] = jnp.zeros_like(acc_ref)

    x = x_ref[...]
    emb = emb_ref[...]
    s = jnp.dot(x, emb, preferred_element_type=jnp.float32)
    x2 = jnp.sum(x ** 2, axis=1, keepdims=True)
    e2 = jnp.sum(emb ** 2, axis=0, keepdims=True)
    d = x2 - 2.0 * s + e2
    m = jnp.min(d, axis=1, keepdims=True)
    eqm = d == m
    iota = lax.broadcasted_iota(jnp.int32, (1, K), 1).astype(jnp.float32)
    idx = jnp.min(jnp.where(eqm, iota, float(K)), axis=1)
    idx_ref[...] = idx.astype(jnp.int32)
    oh = eqm.astype(jnp.float32)
    ones = jnp.ones((1, M), jnp.float32)
    cnt_ref[...] += lax.dot_general(ones, oh, (((1,), (0,)), ((), ())),
                                    preferred_element_type=jnp.float32)
    acc_ref[...] = acc_ref[...] + jnp.sum(m, keepdims=True)

    @pl.when(i == GRID - 1)
    def _fin():
        sse = acc_ref[...]
        loss_ref[...] = (1.25 / (N * D)) * sse
        p = cnt_ref[...] * (1.0 / N)
        ent = -jnp.sum(p * jnp.log(p + 1e-10), keepdims=True)
        perp_ref[...] = jnp.exp(ent)


_tc_call = pl.pallas_call(
    _vq_tc_body,
    grid=(GRID,),
    in_specs=[
        pl.BlockSpec((M, D), lambda i: (i, 0)),
        pl.BlockSpec((D, K), lambda i: (0, 0)),
    ],
    out_specs=[
        pl.BlockSpec((M,), lambda i: (i,)),
        pl.BlockSpec((1, 1), lambda i: (0, 0)),
        pl.BlockSpec((1, 1), lambda i: (0, 0)),
    ],
    out_shape=[
        jax.ShapeDtypeStruct((N,), jnp.int32),
        jax.ShapeDtypeStruct((1, 1), jnp.float32),
        jax.ShapeDtypeStruct((1, 1), jnp.float32),
    ],
    scratch_shapes=[
        pltpu.VMEM((1, K), jnp.float32),
        pltpu.VMEM((1, 1), jnp.float32),
    ],
)


def _sc_gather(table, idx2d):
    mesh = plsc.VectorSubcoreMesh(core_axis_name="c", subcore_axis_name="s")

    @functools.partial(
        pl.kernel,
        mesh=mesh,
        compiler_params=pltpu.CompilerParams(use_tc_tiling_on_sc=False),
        out_type=jax.ShapeDtypeStruct((N, D), jnp.float32),
        scratch_types=[
            pltpu.VMEM((_NCH, _CHUNK), jnp.int32),
            pltpu.VMEM((_BPW, D), jnp.float32),
            pltpu.VMEM_SHARED((K, D), jnp.float32),
            pltpu.SemaphoreType.DMA,
        ],
    )
    def gk(table_hbm, idx_hbm, out_hbm, idx_v, rows_v, table_sh, sem):
        wid = lax.axis_index("s") * 2 + lax.axis_index("c")

        @pl.when(lax.axis_index("s") == 0)
        def _stage():
            pltpu.sync_copy(table_hbm, table_sh)

        pltpu.sync_copy(idx_hbm.at[wid], idx_v)
        plsc.subcore_barrier()
        descs = [
            pltpu.async_copy(table_sh.at[idx_v.at[j]],
                             rows_v.at[pl.ds(j * _CHUNK, _CHUNK)], sem)
            for j in range(_NCH)
        ]
        for dsc in descs:
            dsc.wait()
        pltpu.sync_copy(rows_v, out_hbm.at[pl.ds(wid * _BPW, _BPW)])

    return gk(table, idx2d)


def kernel(x, embedding):
    flat_x = x.reshape(-1, D)
    idx, loss11, perp11 = _tc_call(flat_x, embedding)
    quantized = _sc_gather(embedding.T, idx.reshape(_NW, _NCH, _CHUNK))
    return (
        loss11[0, 0],
        quantized.reshape(x.shape),
        perp11[0, 0],
        idx.reshape(x.shape[:1] + x.shape[2:]),
    )
